# baseline (speedup 1.0000x reference)
import functools

import jax
import jax.numpy as jnp
from jax import lax
from jax.experimental import pallas as pl
from jax.experimental.pallas import tpu as pltpu
from jax.experimental.pallas import tpu_sc as plsc

D_H = 4096
NUM_ASPECTS = 6
BATCH = 4096

_NC = 2
_NS = 16
_NW = _NC * _NS
_BPW = BATCH // _NW
_L = 16
_NGRP = _BPW // _L


@functools.partial(
    pl.kernel,
    mesh=plsc.VectorSubcoreMesh(core_axis_name="c", subcore_axis_name="s"),
    out_type=jax.ShapeDtypeStruct((BATCH * D_H,), jnp.float32),
    scratch_types=[
        pltpu.VMEM((_BPW,), jnp.int32),
        pltpu.VMEM((NUM_ASPECTS * D_H,), jnp.float32),
        pltpu.SemaphoreType.DMA,
    ],
)
def _lookup(idx_hbm, table_hbm, out_hbm, idx_v, table_v, sem):
    wid = lax.axis_index("s") * _NC + lax.axis_index("c")
    base = wid * _BPW
    pltpu.sync_copy(table_hbm, table_v)
    pltpu.sync_copy(idx_hbm.at[pl.ds(base, _BPW)], idx_v)

    copies = []
    for g in range(_NGRP):
        idx16 = idx_v[pl.ds(g * _L, _L)]
        for j in range(_L):
            src = idx16[j] * D_H
            dst = (base + g * _L + j) * D_H
            copies.append(pltpu.make_async_copy(
                table_v.at[pl.ds(src, D_H)], out_hbm.at[pl.ds(dst, D_H)],
                sem))
    for c in copies:
        c.start()
    for c in copies:
        c.wait()


def kernel(aspect_idx, embed_weight):
    out = _lookup(aspect_idx.astype(jnp.int32), embed_weight.reshape(-1))
    return out.reshape(BATCH, D_H)


# speedup vs baseline: 2.1804x; 2.1804x over previous
import functools

import jax
import jax.numpy as jnp
from jax import lax
from jax.experimental import pallas as pl
from jax.experimental.pallas import tpu as pltpu
from jax.experimental.pallas import tpu_sc as plsc

D_H = 4096
NUM_ASPECTS = 6
BATCH = 4096

_SC_ROWS = 2048
_NC = 2
_NS = 16
_NW = _NC * _NS
_BPW = _SC_ROWS // _NW
_L = 16
_NGRP = _BPW // _L

_BLK = 256
_TC_BLK0 = _SC_ROWS // _BLK
_TC_NBLK = (BATCH - _SC_ROWS) // _BLK


@functools.partial(
    pl.kernel,
    mesh=plsc.VectorSubcoreMesh(core_axis_name="c", subcore_axis_name="s"),
    out_type=jax.ShapeDtypeStruct((BATCH, D_H), jnp.float32),
    scratch_types=[
        pltpu.VMEM((_BPW,), jnp.int32),
        pltpu.VMEM((NUM_ASPECTS, D_H), jnp.float32),
        pltpu.SemaphoreType.DMA,
    ],
)
def _sc_lookup(idx_hbm, table_hbm, out_hbm, idx_v, table_v, sem):
    wid = lax.axis_index("s") * _NC + lax.axis_index("c")
    base = wid * _BPW
    pltpu.sync_copy(table_hbm, table_v)
    pltpu.sync_copy(idx_hbm.at[pl.ds(base, _BPW)], idx_v)

    copies = []
    for g in range(_NGRP):
        idx16 = idx_v[pl.ds(g * _L, _L)]
        for j in range(_L):
            sj = idx16[j]
            row = base + g * _L + j
            copies.append(pltpu.make_async_copy(
                table_v.at[pl.ds(sj, 1)], out_hbm.at[pl.ds(row, 1)], sem))
    for c in copies:
        c.start()
    for c in copies:
        c.wait()


def _tc_body(idx_ref, table_ref, part_ref, out_ref):
    del part_ref
    idx = idx_ref[0]
    onehot = (idx == lax.broadcasted_iota(jnp.int32, (1, NUM_ASPECTS), 1)
              ).astype(jnp.float32)
    out_ref[...] = jnp.dot(onehot, table_ref[...],
                           preferred_element_type=jnp.float32)


def kernel(aspect_idx, embed_weight):
    idx = aspect_idx.astype(jnp.int32)
    part = _sc_lookup(idx, embed_weight)
    idx3 = idx.reshape(BATCH // _BLK, _BLK, 1)
    return pl.pallas_call(
        _tc_body,
        grid=(_TC_NBLK,),
        in_specs=[
            pl.BlockSpec((1, _BLK, 1), lambda i: (i + _TC_BLK0, 0, 0)),
            pl.BlockSpec((NUM_ASPECTS, D_H), lambda i: (0, 0)),
            pl.BlockSpec(memory_space=pl.ANY),
        ],
        out_specs=pl.BlockSpec((_BLK, D_H), lambda i: (i + _TC_BLK0, 0)),
        out_shape=jax.ShapeDtypeStruct((BATCH, D_H), jnp.float32),
        input_output_aliases={2: 0},
    )(idx3, embed_weight, part)
